# baseline (speedup 1.0000x reference)
import functools

import jax
import jax.numpy as jnp
from jax import lax
from jax.experimental import pallas as pl
from jax.experimental.pallas import tpu as pltpu
from jax.experimental.pallas import tpu_sc as plsc

B = 262144
K = 256
V = 4
BM = 4096

_NC = 2
_NS = 16
_NW = _NC * _NS
_BPW = B // _NW
_L = 16


def _tc_body(x_ref, tlutT_ref, state_ref):
    x = x_ref[...]
    tT = tlutT_ref[...]
    t2 = jnp.sum(tT * tT, axis=0, keepdims=True)
    xt = lax.dot_general(x, tT, (((1,), (0,)), ((), ())),
                         preferred_element_type=jnp.float32)
    d2 = t2 - 2.0 * xt
    m = jnp.min(d2, axis=1, keepdims=True)
    lanes = lax.broadcasted_iota(jnp.int32, (BM, K), 1)
    state_ref[...] = jnp.min(jnp.where(d2 == m, lanes, K), axis=1,
                             keepdims=True)


def _sc_body(tlutT_hbm, state_hbm, out_hbm, tT_v, idx_v, rows_v):
    wid = lax.axis_index("s") * _NC + lax.axis_index("c")
    pltpu.sync_copy(tlutT_hbm, tT_v)
    pltpu.sync_copy(state_hbm.at[wid], idx_v)
    lane = lax.iota(jnp.int32, _L)

    def _step(i, carry):
        s16 = idx_v[pl.ds(i * _L, _L)]
        pos0 = (i * _L * V) + lane * V
        for j in range(V):
            vals = plsc.load_gather(tT_v, [s16 + (j * K)])
            plsc.store_scatter(rows_v, [pos0 + j], vals)
        return carry

    lax.fori_loop(0, _BPW // _L, _step, 0)
    pltpu.sync_copy(rows_v, out_hbm.at[pl.ds(wid * _BPW * V, _BPW * V)])


def kernel(X, tlut):
    tlutT = tlut.T
    state2d = pl.pallas_call(
        _tc_body,
        grid=(B // BM,),
        in_specs=[
            pl.BlockSpec((BM, V), lambda i: (i, 0)),
            pl.BlockSpec((V, K), lambda i: (0, 0)),
        ],
        out_specs=pl.BlockSpec((BM, 1), lambda i: (i, 0)),
        out_shape=jax.ShapeDtypeStruct((B, 1), jnp.int32),
    )(X, tlutT)

    state3 = jnp.zeros((_NW, _BPW), jnp.int32)
    hat_flat = pl.kernel(
        _sc_body,
        out_type=jax.ShapeDtypeStruct((B * V,), jnp.float32),
        mesh=plsc.VectorSubcoreMesh(core_axis_name="c", subcore_axis_name="s"),
        compiler_params=pltpu.CompilerParams(needs_layout_passes=False),
        scratch_types=[
            pltpu.VMEM((V * K,), jnp.float32),
            pltpu.VMEM((_BPW,), jnp.int32),
            pltpu.VMEM((_BPW * V,), jnp.float32),
        ],
    )(tlutT.reshape(V * K), state3)
    return hat_flat.reshape(B, V), state2d.reshape(B)


# speedup vs baseline: 3.8431x; 3.8431x over previous
import jax
import jax.numpy as jnp
from jax import lax
from jax.experimental import pallas as pl
from jax.experimental.pallas import tpu as pltpu

B = 262144
K = 256
V = 4
BN = 2048


def _tc_body(xt_ref, tlut_ref, hat_ref, state_ref):
    x = xt_ref[...]
    tl = tlut_ref[...]
    t2 = jnp.sum(tl * tl, axis=1, keepdims=True)
    xt = lax.dot_general(tl, x, (((1,), (0,)), ((), ())),
                         preferred_element_type=jnp.float32)
    d2 = t2 - 2.0 * xt
    m = jnp.min(d2, axis=0, keepdims=True)
    sub = lax.broadcasted_iota(jnp.int32, (K, BN), 0)
    idx = jnp.min(jnp.where(d2 == m, sub, K), axis=0,
                  keepdims=True)
    oh = (sub == idx).astype(jnp.float32)
    hat_ref[...] = lax.dot_general(tl, oh, (((0,), (0,)), ((), ())),
                                   preferred_element_type=jnp.float32)
    state_ref[...] = idx[0]


def kernel(X, tlut):
    hatT, state = pl.pallas_call(
        _tc_body,
        grid=(B // BN,),
        in_specs=[
            pl.BlockSpec((V, BN), lambda i: (0, i)),
            pl.BlockSpec((K, V), lambda i: (0, 0)),
        ],
        out_specs=[
            pl.BlockSpec((V, BN), lambda i: (0, i)),
            pl.BlockSpec((BN,), lambda i: (i,)),
        ],
        out_shape=[
            jax.ShapeDtypeStruct((V, B), jnp.float32),
            jax.ShapeDtypeStruct((B,), jnp.int32),
        ],
    )(X.T, tlut)
    return hatT.T, state


# speedup vs baseline: 4.5500x; 1.1839x over previous
import jax
import jax.numpy as jnp
from jax import lax
from jax.experimental import pallas as pl
from jax.experimental.pallas import tpu as pltpu

B = 262144
K = 256
V = 4
BN = 8192


def _tc_body(xt_ref, tlut_ref, hat_ref, state_ref):
    x = xt_ref[...]
    tl = tlut_ref[...]
    t2 = jnp.sum(tl * tl, axis=1, keepdims=True)
    xt = lax.dot_general(tl, x, (((1,), (0,)), ((), ())),
                         preferred_element_type=jnp.float32)
    d2 = t2 - 2.0 * xt
    m = jnp.min(d2, axis=0, keepdims=True)
    sub = lax.broadcasted_iota(jnp.int32, (K, BN), 0)
    idx = jnp.min(jnp.where(d2 == m, sub, K), axis=0,
                  keepdims=True)
    oh = (sub == idx).astype(jnp.float32)
    hat_ref[...] = lax.dot_general(tl, oh, (((0,), (0,)), ((), ())),
                                   preferred_element_type=jnp.float32)
    state_ref[...] = idx[0]


def kernel(X, tlut):
    hatT, state = pl.pallas_call(
        _tc_body,
        grid=(B // BN,),
        in_specs=[
            pl.BlockSpec((V, BN), lambda i: (0, i)),
            pl.BlockSpec((K, V), lambda i: (0, 0)),
        ],
        out_specs=[
            pl.BlockSpec((V, BN), lambda i: (0, i)),
            pl.BlockSpec((BN,), lambda i: (i,)),
        ],
        out_shape=[
            jax.ShapeDtypeStruct((V, B), jnp.float32),
            jax.ShapeDtypeStruct((B,), jnp.int32),
        ],
    )(X.T, tlut)
    return hatT.T, state


# speedup vs baseline: 4.6997x; 1.0329x over previous
import jax
import jax.numpy as jnp
from jax import lax
from jax.experimental import pallas as pl
from jax.experimental.pallas import tpu as pltpu

B = 262144
K = 256
V = 4
BN = 16384


def _tc_body(xt_ref, tlut_ref, hat_ref, state_ref):
    x = xt_ref[...]
    tl = tlut_ref[...]
    t2 = jnp.sum(tl * tl, axis=1, keepdims=True)
    xt = lax.dot_general(tl, x, (((1,), (0,)), ((), ())),
                         preferred_element_type=jnp.float32)
    d2 = t2 - 2.0 * xt
    m = jnp.min(d2, axis=0, keepdims=True)
    sub = lax.broadcasted_iota(jnp.int32, (K, BN), 0)
    idx = jnp.min(jnp.where(d2 == m, sub, K), axis=0,
                  keepdims=True)
    oh = (sub == idx).astype(jnp.float32)
    hat_ref[...] = lax.dot_general(tl, oh, (((0,), (0,)), ((), ())),
                                   preferred_element_type=jnp.float32)
    state_ref[...] = idx[0]


def kernel(X, tlut):
    hatT, state = pl.pallas_call(
        _tc_body,
        grid=(B // BN,),
        in_specs=[
            pl.BlockSpec((V, BN), lambda i: (0, i)),
            pl.BlockSpec((K, V), lambda i: (0, 0)),
        ],
        out_specs=[
            pl.BlockSpec((V, BN), lambda i: (0, i)),
            pl.BlockSpec((BN,), lambda i: (i,)),
        ],
        out_shape=[
            jax.ShapeDtypeStruct((V, B), jnp.float32),
            jax.ShapeDtypeStruct((B,), jnp.int32),
        ],
    )(X.T, tlut)
    return hatT.T, state
